# baseline (speedup 1.0000x reference)
import functools

import jax
import jax.numpy as jnp
from jax.experimental import pallas as pl
from jax.experimental.pallas import tpu as pltpu


def _swiglu_body(x_ref, wg_ref, wu_ref, wd_ref, o_ref):
    h = pl.program_id(1)

    xb = x_ref[...]
    gate = jax.lax.dot_general(
        xb, wg_ref[...].astype(jnp.bfloat16), (((1,), (1,)), ((), ())),
        preferred_element_type=jnp.float32)
    up = jax.lax.dot_general(
        xb, wu_ref[...].astype(jnp.bfloat16), (((1,), (1,)), ((), ())),
        preferred_element_type=jnp.float32)
    z = (gate * jax.nn.sigmoid(gate) * up).astype(jnp.bfloat16)

    @pl.when(h == 0)
    def _init():
        o_ref[...] = jnp.zeros_like(o_ref)

    d = o_ref.shape[1]
    n_chunks = 8
    cb = d // n_chunks
    for j in range(n_chunks):
        wdj = wd_ref[pl.ds(j * cb, cb), :].astype(jnp.bfloat16)
        cj = jax.lax.dot_general(
            z, wdj, (((1,), (1,)), ((), ())),
            preferred_element_type=jnp.float32)
        o_ref[:, pl.ds(j * cb, cb)] += cj


@functools.partial(jax.jit, static_argnames=("bm", "bh"))
def _swiglu(x2d, Wg, Wu, Wd, bm=2048, bh=256):
    m, d = x2d.shape
    hidden = Wg.shape[0]
    grid = (m // bm, hidden // bh)
    return pl.pallas_call(
        _swiglu_body,
        grid=grid,
        in_specs=[
            pl.BlockSpec((bm, d), lambda i, h: (i, 0),
                         pipeline_mode=pl.Buffered(buffer_count=1)),
            pl.BlockSpec((bh, d), lambda i, h: (h, 0)),
            pl.BlockSpec((bh, d), lambda i, h: (h, 0)),
            pl.BlockSpec((d, bh), lambda i, h: (0, h)),
        ],
        out_specs=pl.BlockSpec((bm, d), lambda i, h: (i, 0),
                               pipeline_mode=pl.Buffered(buffer_count=1)),
        out_shape=jax.ShapeDtypeStruct((m, d), jnp.float32),
        compiler_params=pltpu.CompilerParams(
            dimension_semantics=("arbitrary", "arbitrary"),
        ),
    )(x2d, Wg, Wu, Wd)


def kernel(x, Wg, Wu, Wd):
    shape = x.shape
    d_model = shape[-1]
    x2d = x.reshape(-1, d_model).astype(jnp.bfloat16)
    out = _swiglu(x2d, Wg, Wu, Wd)
    return out.reshape(shape)


# speedup vs baseline: 1.0486x; 1.0486x over previous
import functools

import jax
import jax.numpy as jnp
from jax.experimental import pallas as pl
from jax.experimental.pallas import tpu as pltpu


def _swiglu_body(x_ref, wg_ref, wu_ref, wd_ref, o_ref, z_ref):
    h = pl.program_id(1)

    xb = x_ref[...].astype(jnp.bfloat16)
    gate = jax.lax.dot_general(
        xb, wg_ref[...].astype(jnp.bfloat16), (((1,), (1,)), ((), ())),
        preferred_element_type=jnp.float32)
    up = jax.lax.dot_general(
        xb, wu_ref[...].astype(jnp.bfloat16), (((1,), (1,)), ((), ())),
        preferred_element_type=jnp.float32)
    bh = gate.shape[1]
    z_ref[:, pl.ds((h % 2) * bh, bh)] = (
        gate * jax.nn.sigmoid(gate) * up).astype(jnp.bfloat16)

    @pl.when(h == 0)
    def _init():
        o_ref[...] = jnp.zeros_like(o_ref)

    @pl.when(h % 2 == 1)
    def _down():
        z = z_ref[...]
        d = o_ref.shape[1]
        n_chunks = 8
        cb = d // n_chunks
        for j in range(n_chunks):
            wdj = wd_ref[pl.ds(j * cb, cb), :].astype(jnp.bfloat16)
            cj = jax.lax.dot_general(
                z, wdj, (((1,), (1,)), ((), ())),
                preferred_element_type=jnp.float32)
            o_ref[:, pl.ds(j * cb, cb)] += cj


@functools.partial(jax.jit, static_argnames=("bm", "bh"))
def _swiglu(x2d, Wg, Wu, Wd, bm=2048, bh=256):
    m, d = x2d.shape
    hidden = Wg.shape[0]
    grid = (m // bm, hidden // bh)
    return pl.pallas_call(
        _swiglu_body,
        grid=grid,
        in_specs=[
            pl.BlockSpec((bm, d), lambda i, h: (i, 0),
                         pipeline_mode=pl.Buffered(buffer_count=1)),
            pl.BlockSpec((bh, d), lambda i, h: (h, 0)),
            pl.BlockSpec((bh, d), lambda i, h: (h, 0)),
            pl.BlockSpec((d, 2 * bh), lambda i, h: (0, h // 2)),
        ],
        out_specs=pl.BlockSpec((bm, d), lambda i, h: (i, 0),
                               pipeline_mode=pl.Buffered(buffer_count=1)),
        out_shape=jax.ShapeDtypeStruct((m, d), jnp.float32),
        scratch_shapes=[
            pltpu.VMEM((bm, 2 * bh), jnp.bfloat16),
        ],
        compiler_params=pltpu.CompilerParams(
            dimension_semantics=("arbitrary", "arbitrary"),
        ),
    )(x2d, Wg, Wu, Wd)


def kernel(x, Wg, Wu, Wd):
    shape = x.shape
    d_model = shape[-1]
    x2d = x.reshape(-1, d_model)
    out = _swiglu(x2d, Wg, Wu, Wd)
    return out.reshape(shape)
